# baseline (device time: 352692 ns/iter reference)
import jax
import jax.numpy as jnp
from jax import lax
from jax.experimental import pallas as pl
from jax.experimental.pallas import tpu as pltpu

N_DEV = 8
N_BLOCKS = 4
SUB = 4
N_HOPS = 2 * (N_DEV - 1)
BOOT = N_DEV - 1


def kernel(x, w_mat, scale_x, scale_w):
    m_total, k_per = x.shape
    _, n = w_mat.shape
    m_per = m_total // N_DEV
    n_blk = n // N_BLOCKS
    n_sub = n_blk // SUB

    x8 = x.astype(jnp.float8_e5m2)
    w8 = w_mat.astype(jnp.float8_e5m2)

    def body(x_ref, w_ref, sx_ref, sw_ref, out_ref,
             send_cw, comm_cw, ssem_cw, rsem_cw, credit_cw,
             send_ccw, comm_ccw, ssem_ccw, rsem_ccw, credit_ccw):
        my = lax.axis_index("i")
        left = (my - 1) % N_DEV
        right = (my + 1) % N_DEV

        barrier = pltpu.get_barrier_semaphore()
        for nbr in (left, right):
            pl.semaphore_signal(barrier, inc=1, device_id=(nbr,),
                                device_id_type=pl.DeviceIdType.MESH)
        pl.semaphore_wait(barrier, 2)

        scale = sx_ref[0] * sw_ref[0]

        def gemm(c, blk):
            xa = x_ref[pl.ds(c * m_per, m_per), :]
            wb = w_ref[:, blk * n_blk:(blk + 1) * n_blk]
            return lax.dot_general(xa, wb, (((1,), (0,)), ((), ())),
                                   preferred_element_type=jnp.float32)

        dirs = [
            dict(send_buf=send_cw, comm_buf=comm_cw, ssem=ssem_cw,
                 rsem=rsem_cw, credit=credit_cw, to=right, frm=left,
                 blk0=0, sign=1),
            dict(send_buf=send_ccw, comm_buf=comm_ccw, ssem=ssem_ccw,
                 rsem=rsem_ccw, credit=credit_ccw, to=left, frm=right,
                 blk0=2, sign=-1),
        ]

        def rdma(d, h, p, target):
            sl = h % 2
            return pltpu.make_async_remote_copy(
                src_ref=d["send_buf"].at[sl, p],
                dst_ref=d["comm_buf"].at[sl, p],
                send_sem=d["ssem"].at[sl, p],
                recv_sem=d["rsem"].at[sl, p],
                device_id=(target,),
                device_id_type=pl.DeviceIdType.MESH,
            )

        def do_send(d, h, p, value):
            if h >= 2:
                pl.semaphore_wait(d["credit"], 1)
                rdma(d, h, p, d["to"]).wait_send()
            d["send_buf"][h % 2, p] = value.astype(jnp.bfloat16)
            rdma(d, h, p, d["to"]).start()

        def consume(d, h_recv, p, part):
            rdma(d, h_recv, p, d["frm"]).wait_recv()
            return d["comm_buf"][h_recv % 2, p].astype(jnp.float32) + part

        def credit_back(d, h_recv):
            if h_recv <= N_HOPS - 3:
                pl.semaphore_signal(d["credit"], inc=1,
                                    device_id=(d["frm"],),
                                    device_id_type=pl.DeviceIdType.MESH)

        def epilogue(d, h_recv, blk, p, part):
            acc = consume(d, h_recv, p, part)
            y = acc * scale
            lo = blk * n_blk + p * n_sub
            out_ref[:, lo:lo + n_sub] = (
                y / (1.0 + jnp.exp(-jnp.clip(y, -60.0, 60.0))))
            credit_back(d, h_recv)

        def step_parts(s, ring):
            full = []
            for d in dirs:
                c = (my - d["sign"] * (1 + s)) % N_DEV
                full.append(gemm(c, d["blk0"] + ring))
            return [[f[:, p * n_sub:(p + 1) * n_sub] for f in full]
                    for p in range(SUB)]

        for t in range(7):
            parts = step_parts(t, 0)
            for p in range(SUB):
                for di, d in enumerate(dirs):
                    part = parts[p][di]
                    acc = part if t == 0 else consume(d, t - 1, p, part)
                    do_send(d, t, p, acc)
                    if t > 0:
                        credit_back(d, t - 1)

        boot_parts = step_parts(0, 1)
        for p in range(SUB):
            for di, d in enumerate(dirs):
                do_send(d, BOOT, p, boot_parts[p][di])
        epi_parts = step_parts(7, 0)
        for p in range(SUB):
            for di, d in enumerate(dirs):
                epilogue(d, BOOT - 1, d["blk0"], p, epi_parts[p][di])

        for t in range(8, 14):
            parts = step_parts(t - 7, 1)
            for p in range(SUB):
                for di, d in enumerate(dirs):
                    acc = consume(d, t - 1, p, parts[p][di])
                    do_send(d, t, p, acc)
                    credit_back(d, t - 1)

        epi_parts = step_parts(7, 1)
        for p in range(SUB):
            for di, d in enumerate(dirs):
                epilogue(d, N_HOPS - 1, d["blk0"] + 1, p, epi_parts[p][di])

        for d in dirs:
            for h in (N_HOPS - 2, N_HOPS - 1):
                for p in range(SUB):
                    rdma(d, h, p, d["to"]).wait_send()

    comm_shape = (2, SUB, m_per, n_sub)
    return pl.pallas_call(
        body,
        out_shape=jax.ShapeDtypeStruct((m_per, n), jnp.float32),
        in_specs=[
            pl.BlockSpec(memory_space=pltpu.VMEM),
            pl.BlockSpec(memory_space=pltpu.VMEM),
            pl.BlockSpec(memory_space=pltpu.SMEM),
            pl.BlockSpec(memory_space=pltpu.SMEM),
        ],
        out_specs=pl.BlockSpec(memory_space=pltpu.VMEM),
        scratch_shapes=[
            pltpu.VMEM(comm_shape, jnp.bfloat16),
            pltpu.VMEM(comm_shape, jnp.bfloat16),
            pltpu.SemaphoreType.DMA((2, SUB)),
            pltpu.SemaphoreType.DMA((2, SUB)),
            pltpu.SemaphoreType.REGULAR,
            pltpu.VMEM(comm_shape, jnp.bfloat16),
            pltpu.VMEM(comm_shape, jnp.bfloat16),
            pltpu.SemaphoreType.DMA((2, SUB)),
            pltpu.SemaphoreType.DMA((2, SUB)),
            pltpu.SemaphoreType.REGULAR,
        ],
        compiler_params=pltpu.CompilerParams(
            collective_id=0, vmem_limit_bytes=int(41.5 * 1024 * 1024)),
    )(x8, w8, scale_x, scale_w)


# device time: 231607 ns/iter; 1.5228x vs baseline; 1.5228x over previous
import jax
import jax.numpy as jnp
from jax import lax
from jax.experimental import pallas as pl
from jax.experimental.pallas import tpu as pltpu

N_DEV = 8
N_HOPS = N_DEV - 1
WSLOTS = 4
CW_XHOPS = 4
CCW_XHOPS = 3


def kernel(x, w_mat, scale_x, scale_w):
    m_total, k_per = x.shape
    _, n = w_mat.shape
    m_per = m_total // N_DEV
    half = n // 2
    n_acc = 2048

    x8 = x.astype(jnp.float8_e5m2)
    w8 = w_mat.astype(jnp.float8_e5m2)

    def body(x_ref, w_ref, sx_ref, sw_ref, out_ref,
             wbuf_cw, wbuf_ccw, wssem_cw, wrsem_cw, wssem_ccw, wrsem_ccw,
             credit_cw, credit_ccw,
             xstage_cw, xstage_ccw,
             xb_cw1, xb_cw2, xb_cw3, xb_cw4, xb_ccw1, xb_ccw2, xb_ccw3,
             xssem_cw, xrsem_cw, xssem_ccw, xrsem_ccw):
        my = lax.axis_index("i")
        left = (my - 1) % N_DEV
        right = (my + 1) % N_DEV

        barrier = pltpu.get_barrier_semaphore()
        for nbr in (left, right):
            pl.semaphore_signal(barrier, inc=1, device_id=(nbr,),
                                device_id_type=pl.DeviceIdType.MESH)
        pl.semaphore_wait(barrier, 2)

        scale = sx_ref[0] * sw_ref[0]
        xb_cw = [xb_cw1, xb_cw2, xb_cw3, xb_cw4]
        xb_ccw = [xb_ccw1, xb_ccw2, xb_ccw3]

        dirs = [
            dict(sign=1, to=right, frm=left, off=0, wbuf=wbuf_cw,
                 wssem=wssem_cw, wrsem=wrsem_cw, credit=credit_cw,
                 xstage=xstage_cw, xb=xb_cw, xhops=CW_XHOPS,
                 xssem=xssem_cw, xrsem=xrsem_cw),
            dict(sign=-1, to=left, frm=right, off=half, wbuf=wbuf_ccw,
                 wssem=wssem_ccw, wrsem=wrsem_ccw, credit=credit_ccw,
                 xstage=xstage_ccw, xb=xb_ccw, xhops=CCW_XHOPS,
                 xssem=xssem_ccw, xrsem=xrsem_ccw),
        ]

        def w_rdma(d, h, src_ref, target):
            return pltpu.make_async_remote_copy(
                src_ref=src_ref,
                dst_ref=d["wbuf"].at[(h - 1) % WSLOTS],
                send_sem=d["wssem"].at[h - 1],
                recv_sem=d["wrsem"].at[h - 1],
                device_id=(target,),
                device_id_type=pl.DeviceIdType.MESH,
            )

        def x_rdma(d, h, src_ref, target):
            return pltpu.make_async_remote_copy(
                src_ref=src_ref,
                dst_ref=d["xb"][h - 1].at[:],
                send_sem=d["xssem"].at[h - 1],
                recv_sem=d["xrsem"].at[h - 1],
                device_id=(target,),
                device_id_type=pl.DeviceIdType.MESH,
            )

        def xpiece(di, h):
            d = dirs[di]
            if h <= d["xhops"]:
                return d["xb"][h - 1][0]
            other = dirs[1 - di]
            return other["xb"][(N_DEV - h) - 1][0]

        def accumulate(di, h):
            d = dirs[di]
            xp = xpiece(di, h)
            wv = d["wbuf"][(h - 1) % WSLOTS]
            for c in range(half // n_acc):
                lo = d["off"] + c * n_acc
                dot = lax.dot_general(
                    xp, wv[:, c * n_acc:(c + 1) * n_acc],
                    (((1,), (0,)), ((), ())),
                    preferred_element_type=jnp.float32)
                out_ref[:, lo:lo + n_acc] = out_ref[:, lo:lo + n_acc] + dot

        for d in dirs:
            for q in range(d["xhops"]):
                g = (my + d["sign"] * (1 + q)) % N_DEV
                d["xstage"][q] = x_ref[pl.ds(g * m_per, m_per), :]
        for d in dirs:
            w_rdma(d, 1, w_ref.at[:, d["off"]:d["off"] + half],
                   d["to"]).start()
            x_rdma(d, 1, d["xstage"].at[:], d["to"]).start()

        xme = x_ref[pl.ds(my * m_per, m_per), :]
        for c in range(n // n_acc):
            dot = lax.dot_general(
                xme, w_ref[:, c * n_acc:(c + 1) * n_acc],
                (((1,), (0,)), ((), ())),
                preferred_element_type=jnp.float32)
            out_ref[:, c * n_acc:(c + 1) * n_acc] = dot

        for h in range(1, N_HOPS + 1):
            for d in dirs:
                w_rdma(d, h, d["wbuf"].at[(h - 1) % WSLOTS],
                       d["frm"]).wait_recv()
                if h < N_HOPS:
                    if h + 1 > WSLOTS:
                        pl.semaphore_wait(d["credit"], 1)
                    w_rdma(d, h + 1, d["wbuf"].at[(h - 1) % WSLOTS],
                           d["to"]).start()
                if h <= d["xhops"]:
                    x_rdma(d, h, d["xb"][h - 1].at[:],
                           d["frm"]).wait_recv()
                    if h < d["xhops"]:
                        npc = d["xhops"] - h
                        x_rdma(d, h + 1, d["xb"][h - 1].at[1:1 + npc],
                               d["to"]).start()
            for di in range(2):
                accumulate(di, h)
            if 3 <= h <= N_HOPS - 2:
                for d in dirs:
                    w_rdma(d, h - 1, d["wbuf"].at[(h - 2) % WSLOTS],
                           d["to"]).wait_send()
                    pl.semaphore_signal(d["credit"], inc=1,
                                        device_id=(d["frm"],),
                                        device_id_type=pl.DeviceIdType.MESH)

        for c in range(n // n_acc):
            y = out_ref[:, c * n_acc:(c + 1) * n_acc] * scale
            out_ref[:, c * n_acc:(c + 1) * n_acc] = (
                y / (1.0 + jnp.exp(-jnp.clip(y, -60.0, 60.0))))

        for d in dirs:
            for h in (1, 5, 6, 7):
                src = (w_ref.at[:, d["off"]:d["off"] + half] if h == 1
                       else d["wbuf"].at[(h - 2) % WSLOTS])
                w_rdma(d, h, src, d["to"]).wait_send()
            for h in range(1, d["xhops"] + 1):
                if h == 1:
                    src = d["xstage"].at[:]
                else:
                    npc = d["xhops"] - (h - 1)
                    src = d["xb"][h - 2].at[1:1 + npc]
                x_rdma(d, h, src, d["to"]).wait_send()

    e5 = jnp.float8_e5m2
    return pl.pallas_call(
        body,
        out_shape=jax.ShapeDtypeStruct((m_per, n), jnp.float32),
        in_specs=[
            pl.BlockSpec(memory_space=pltpu.VMEM),
            pl.BlockSpec(memory_space=pltpu.VMEM),
            pl.BlockSpec(memory_space=pltpu.SMEM),
            pl.BlockSpec(memory_space=pltpu.SMEM),
        ],
        out_specs=pl.BlockSpec(memory_space=pltpu.VMEM),
        scratch_shapes=[
            pltpu.VMEM((WSLOTS, k_per, half), e5),
            pltpu.VMEM((WSLOTS, k_per, half), e5),
            pltpu.SemaphoreType.DMA((N_HOPS,)),
            pltpu.SemaphoreType.DMA((N_HOPS,)),
            pltpu.SemaphoreType.DMA((N_HOPS,)),
            pltpu.SemaphoreType.DMA((N_HOPS,)),
            pltpu.SemaphoreType.REGULAR,
            pltpu.SemaphoreType.REGULAR,
            pltpu.VMEM((CW_XHOPS, m_per, k_per), e5),
            pltpu.VMEM((CCW_XHOPS, m_per, k_per), e5),
            pltpu.VMEM((4, m_per, k_per), e5),
            pltpu.VMEM((3, m_per, k_per), e5),
            pltpu.VMEM((2, m_per, k_per), e5),
            pltpu.VMEM((1, m_per, k_per), e5),
            pltpu.VMEM((3, m_per, k_per), e5),
            pltpu.VMEM((2, m_per, k_per), e5),
            pltpu.VMEM((1, m_per, k_per), e5),
            pltpu.SemaphoreType.DMA((CW_XHOPS,)),
            pltpu.SemaphoreType.DMA((CW_XHOPS,)),
            pltpu.SemaphoreType.DMA((CCW_XHOPS,)),
            pltpu.SemaphoreType.DMA((CCW_XHOPS,)),
        ],
        compiler_params=pltpu.CompilerParams(
            collective_id=0, vmem_limit_bytes=int(41.5 * 1024 * 1024)),
    )(x8, w8, scale_x, scale_w)


# device time: 231530 ns/iter; 1.5233x vs baseline; 1.0003x over previous
import jax
import jax.numpy as jnp
from jax import lax
from jax.experimental import pallas as pl
from jax.experimental.pallas import tpu as pltpu

N_DEV = 8
N_HOPS = N_DEV - 1
WSLOTS = 4
CW_XHOPS = 4
CCW_XHOPS = 3


def kernel(x, w_mat, scale_x, scale_w):
    m_total, k_per = x.shape
    _, n = w_mat.shape
    m_per = m_total // N_DEV
    half = n // 2
    n_acc = 2048

    x8 = x.astype(jnp.float8_e5m2)
    w8 = w_mat.astype(jnp.float8_e5m2)

    def body(x_ref, w_ref, sx_ref, sw_ref, out_ref,
             wbuf_cw, wbuf_ccw, wssem_cw, wrsem_cw, wssem_ccw, wrsem_ccw,
             credit_cw, credit_ccw,
             xstage_cw, xstage_ccw,
             xb_cw1, xb_cw2, xb_cw3, xb_cw4, xb_ccw1, xb_ccw2, xb_ccw3,
             xssem_cw, xrsem_cw, xssem_ccw, xrsem_ccw):
        my = lax.axis_index("i")
        left = (my - 1) % N_DEV
        right = (my + 1) % N_DEV

        barrier = pltpu.get_barrier_semaphore()
        for nbr in (left, right):
            pl.semaphore_signal(barrier, inc=1, device_id=(nbr,),
                                device_id_type=pl.DeviceIdType.MESH)
        pl.semaphore_wait(barrier, 2)

        scale = sx_ref[0] * sw_ref[0]
        xb_cw = [xb_cw1, xb_cw2, xb_cw3, xb_cw4]
        xb_ccw = [xb_ccw1, xb_ccw2, xb_ccw3]

        dirs = [
            dict(sign=1, to=right, frm=left, off=0, wbuf=wbuf_cw,
                 wssem=wssem_cw, wrsem=wrsem_cw, credit=credit_cw,
                 xstage=xstage_cw, xb=xb_cw, xhops=CW_XHOPS,
                 xssem=xssem_cw, xrsem=xrsem_cw),
            dict(sign=-1, to=left, frm=right, off=half, wbuf=wbuf_ccw,
                 wssem=wssem_ccw, wrsem=wrsem_ccw, credit=credit_ccw,
                 xstage=xstage_ccw, xb=xb_ccw, xhops=CCW_XHOPS,
                 xssem=xssem_ccw, xrsem=xrsem_ccw),
        ]

        def w_rdma(d, h, src_ref, target):
            return pltpu.make_async_remote_copy(
                src_ref=src_ref,
                dst_ref=d["wbuf"].at[(h - 1) % WSLOTS],
                send_sem=d["wssem"].at[h - 1],
                recv_sem=d["wrsem"].at[h - 1],
                device_id=(target,),
                device_id_type=pl.DeviceIdType.MESH,
            )

        def x_rdma(d, h, src_ref, target):
            return pltpu.make_async_remote_copy(
                src_ref=src_ref,
                dst_ref=d["xb"][h - 1].at[:],
                send_sem=d["xssem"].at[h - 1],
                recv_sem=d["xrsem"].at[h - 1],
                device_id=(target,),
                device_id_type=pl.DeviceIdType.MESH,
            )

        def xpiece(di, h):
            d = dirs[di]
            if h <= d["xhops"]:
                return d["xb"][h - 1][0]
            other = dirs[1 - di]
            return other["xb"][(N_DEV - h) - 1][0]

        def accumulate(di, h):
            d = dirs[di]
            xp = xpiece(di, h)
            wv = d["wbuf"][(h - 1) % WSLOTS]
            for c in range(half // n_acc):
                lo = d["off"] + c * n_acc
                dot = lax.dot_general(
                    xp, wv[:, c * n_acc:(c + 1) * n_acc],
                    (((1,), (0,)), ((), ())),
                    preferred_element_type=jnp.float32)
                acc = out_ref[:, lo:lo + n_acc] + dot
                if h == N_HOPS:
                    y = acc * scale
                    acc = y / (1.0 + jnp.exp(-jnp.clip(y, -60.0, 60.0)))
                out_ref[:, lo:lo + n_acc] = acc

        for d in dirs:
            for q in range(d["xhops"]):
                g = (my + d["sign"] * (1 + q)) % N_DEV
                d["xstage"][q] = x_ref[pl.ds(g * m_per, m_per), :]
        for d in dirs:
            w_rdma(d, 1, w_ref.at[:, d["off"]:d["off"] + half],
                   d["to"]).start()
            x_rdma(d, 1, d["xstage"].at[:], d["to"]).start()

        xme = x_ref[pl.ds(my * m_per, m_per), :]
        for c in range(n // n_acc):
            dot = lax.dot_general(
                xme, w_ref[:, c * n_acc:(c + 1) * n_acc],
                (((1,), (0,)), ((), ())),
                preferred_element_type=jnp.float32)
            out_ref[:, c * n_acc:(c + 1) * n_acc] = dot

        for h in range(1, N_HOPS + 1):
            for d in dirs:
                w_rdma(d, h, d["wbuf"].at[(h - 1) % WSLOTS],
                       d["frm"]).wait_recv()
                if h < N_HOPS:
                    if h + 1 > WSLOTS:
                        pl.semaphore_wait(d["credit"], 1)
                    w_rdma(d, h + 1, d["wbuf"].at[(h - 1) % WSLOTS],
                           d["to"]).start()
                if h <= d["xhops"]:
                    x_rdma(d, h, d["xb"][h - 1].at[:],
                           d["frm"]).wait_recv()
                    if h < d["xhops"]:
                        npc = d["xhops"] - h
                        x_rdma(d, h + 1, d["xb"][h - 1].at[1:1 + npc],
                               d["to"]).start()
            for di in range(2):
                accumulate(di, h)
            if 3 <= h <= N_HOPS - 2:
                for d in dirs:
                    w_rdma(d, h - 1, d["wbuf"].at[(h - 2) % WSLOTS],
                           d["to"]).wait_send()
                    pl.semaphore_signal(d["credit"], inc=1,
                                        device_id=(d["frm"],),
                                        device_id_type=pl.DeviceIdType.MESH)

        for d in dirs:
            for h in (1, 5, 6, 7):
                src = (w_ref.at[:, d["off"]:d["off"] + half] if h == 1
                       else d["wbuf"].at[(h - 2) % WSLOTS])
                w_rdma(d, h, src, d["to"]).wait_send()
            for h in range(1, d["xhops"] + 1):
                if h == 1:
                    src = d["xstage"].at[:]
                else:
                    npc = d["xhops"] - (h - 1)
                    src = d["xb"][h - 2].at[1:1 + npc]
                x_rdma(d, h, src, d["to"]).wait_send()

    e5 = jnp.float8_e5m2
    return pl.pallas_call(
        body,
        out_shape=jax.ShapeDtypeStruct((m_per, n), jnp.float32),
        in_specs=[
            pl.BlockSpec(memory_space=pltpu.VMEM),
            pl.BlockSpec(memory_space=pltpu.VMEM),
            pl.BlockSpec(memory_space=pltpu.SMEM),
            pl.BlockSpec(memory_space=pltpu.SMEM),
        ],
        out_specs=pl.BlockSpec(memory_space=pltpu.VMEM),
        scratch_shapes=[
            pltpu.VMEM((WSLOTS, k_per, half), e5),
            pltpu.VMEM((WSLOTS, k_per, half), e5),
            pltpu.SemaphoreType.DMA((N_HOPS,)),
            pltpu.SemaphoreType.DMA((N_HOPS,)),
            pltpu.SemaphoreType.DMA((N_HOPS,)),
            pltpu.SemaphoreType.DMA((N_HOPS,)),
            pltpu.SemaphoreType.REGULAR,
            pltpu.SemaphoreType.REGULAR,
            pltpu.VMEM((CW_XHOPS, m_per, k_per), e5),
            pltpu.VMEM((CCW_XHOPS, m_per, k_per), e5),
            pltpu.VMEM((4, m_per, k_per), e5),
            pltpu.VMEM((3, m_per, k_per), e5),
            pltpu.VMEM((2, m_per, k_per), e5),
            pltpu.VMEM((1, m_per, k_per), e5),
            pltpu.VMEM((3, m_per, k_per), e5),
            pltpu.VMEM((2, m_per, k_per), e5),
            pltpu.VMEM((1, m_per, k_per), e5),
            pltpu.SemaphoreType.DMA((CW_XHOPS,)),
            pltpu.SemaphoreType.DMA((CW_XHOPS,)),
            pltpu.SemaphoreType.DMA((CCW_XHOPS,)),
            pltpu.SemaphoreType.DMA((CCW_XHOPS,)),
        ],
        compiler_params=pltpu.CompilerParams(
            collective_id=0, vmem_limit_bytes=int(41.5 * 1024 * 1024)),
    )(x8, w8, scale_x, scale_w)
